# baseline (speedup 1.0000x reference)
import jax
import jax.numpy as jnp
from jax import lax
from jax.experimental import pallas as pl
from jax.experimental.pallas import tpu as pltpu

C_CONST = 10.0


def _dot_t(x, w):
    return lax.dot_general(x, w, (((1,), (1,)), ((), ())),
                           preferred_element_type=jnp.float32)


def _score(u2d, vp_col):
    return lax.dot_general(u2d, vp_col, (((1,), (0,)), ((), ())))


def _decode_tc(cce_b, cc0, cc1, cc2, cc3, costs2, G_t, init_w, Wh, bh, Wv,
               bv, Wq, bq, Wr, br, v_ptr, interpret=False):
    B, n4, NE = cce_b.shape
    item = G_t.shape[0]
    VC = 8

    def body(cce_ref, cc0_ref, cc1_ref, cc2_ref, cc3_ref, costs_ref, G_ref,
             iw_ref, Wh_ref, bh_ref, Wv_ref, bv_ref, Wq_ref, bq_ref, Wr_ref,
             br_ref, vp_ref, lp_ref, rew_ref, act_ref, qall_ref, T_ref):
        cce = cce_ref[...]
        Wv_ = Wv_ref[...]
        Wq_ = Wq_ref[...]
        bq_ = bq_ref[...]
        bv_ = bv_ref[...]
        vp = vp_ref[...]
        cc0 = cc0_ref[...]
        cc1 = cc1_ref[...]
        cc2 = cc2_ref[...]
        cc3 = cc3_ref[...]
        costs = costs_ref[...]

        h_mean = jnp.mean(cce, axis=1)
        h_bar = _dot_t(h_mean, Wh_ref[...]) + bh_ref[...]
        ref_proj = (_dot_t(cce.reshape(B * n4, NE), Wr_ref[...])
                    + br_ref[...]).reshape(B, n4, NE)
        cce_row0 = cce[:, 0, :]

        it_n4 = lax.broadcasted_iota(jnp.int32, (B, n4), 1)
        it_ne = lax.broadcasted_iota(jnp.int32, (B, NE), 1)
        it_item = lax.broadcasted_iota(jnp.int32, (B, item), 1)

        query0 = h_bar + _dot_t(iw_ref[...], Wv_) + bv_
        q0 = _dot_t(query0, Wq_) + bq_
        u0 = C_CONST * jnp.tanh(q0[:, None, :] + ref_proj)
        s0 = _score(u0.reshape(B * n4, NE), vp).reshape(B, n4)
        m0 = jnp.max(s0, axis=1, keepdims=True)
        e0 = jnp.exp(s0 - m0)
        p0 = e0 / jnp.sum(e0, axis=1, keepdims=True)
        z0 = jnp.log(p0 + 1e-20) + G_ref[0]
        zmax0 = jnp.max(z0, axis=1, keepdims=True)
        idx0 = jnp.min(jnp.where(z0 == zmax0, it_n4, n4 + 1), axis=1,
                       keepdims=True)
        oh0 = (it_n4 == idx0).astype(jnp.float32)
        lp = jnp.log(jnp.sum(p0 * oh0, axis=1, keepdims=True) + 1e-20)
        mask = jnp.where((it_n4 >> 2) == (idx0 >> 2), 1,
                         jnp.zeros((B, n4), jnp.int32))
        sptx = jnp.sum(cc2 * oh0, axis=1, keepdims=True)
        spty = jnp.sum(cc3 * oh0, axis=1, keepdims=True)
        cprev = jnp.sum(costs * oh0, axis=1, keepdims=True)
        oh_prev = (it_ne == jnp.minimum(idx0, NE - 1)).astype(jnp.float32)
        aval = jnp.sum(cce_row0 * oh_prev, axis=1, keepdims=True)
        acts = idx0 * (it_item == 0).astype(jnp.int32)

        concat = jnp.concatenate(
            [jnp.broadcast_to(aval[:, :, None], (B, NE, NE)),
             jnp.broadcast_to(cce_row0[:, :, None], (B, NE, NE))],
            axis=2)
        h_rest = _dot_t(concat.reshape(B * NE, 2 * NE), Wv_).reshape(
            B, NE, NE) + bv_
        qall_ref[...] = _dot_t(
            (h_bar[:, None, :] + h_rest).reshape(B * NE, NE),
            Wq_).reshape(B, NE, NE) + bq_

        def tstep(k, _):
            qa = qall_ref[:, pl.ds(k * VC, VC), :]
            u = C_CONST * jnp.tanh(qa[:, :, None, :] + ref_proj[:, None])
            T_ref[:, pl.ds(k * VC, VC), :] = _score(
                u.reshape(B * VC * n4, NE), vp).reshape(B, VC, n4)
            return 0

        lax.fori_loop(0, NE // VC, tstep, 0)
        T = T_ref[...]

        def step(i, carry):
            (oh_prev, mask, sptx, spty, cprev, lp, rew, acts) = carry
            s = lax.dot_general(
                oh_prev[:, None, :], T, (((2,), (1,)), ((0,), (0,))),
                precision=lax.Precision.HIGHEST)[:, 0, :]
            s = jnp.where(mask == 1, -1e9, s)
            m = jnp.max(s, axis=1, keepdims=True)
            e = jnp.exp(s - m)
            p = e / jnp.sum(e, axis=1, keepdims=True)
            logits = jnp.log(p + 1e-20)
            z = logits + G_ref[i]
            zmax = jnp.max(z, axis=1, keepdims=True)
            idx = jnp.min(jnp.where(z == zmax, it_n4, n4 + 1), axis=1,
                          keepdims=True)
            oh = (it_n4 == idx).astype(jnp.float32)
            p_sel = jnp.sum(p * oh, axis=1, keepdims=True)
            lp = lp + jnp.log(p_sel + 1e-20)
            mask = jnp.where((it_n4 >> 2) == (idx >> 2), 1, mask)
            ex = jnp.sum(cc0 * oh, axis=1, keepdims=True)
            ey = jnp.sum(cc1 * oh, axis=1, keepdims=True)
            ext = jnp.sqrt((ex - sptx) ** 2 + (ey - spty) ** 2)
            ccur = jnp.sum(costs * oh, axis=1, keepdims=True)
            rew = rew + ext + cprev + ccur
            sptx = jnp.sum(cc2 * oh, axis=1, keepdims=True)
            spty = jnp.sum(cc3 * oh, axis=1, keepdims=True)
            oh_prev = (it_ne == jnp.minimum(idx, NE - 1)).astype(
                jnp.float32)
            acts = acts + idx * (it_item == i).astype(jnp.int32)
            return (oh_prev, mask, sptx, spty, ccur, lp, rew, acts)

        zB1 = jnp.zeros((B, 1), jnp.float32)
        carry0 = (oh_prev, mask, sptx, spty, cprev, lp, zB1, acts)
        (_, _, _, _, _, lp, rew, acts) = lax.fori_loop(1, item, step,
                                                       carry0)
        lp_ref[...] = lp
        rew_ref[...] = rew
        act_ref[...] = acts

    out_shape = (
        jax.ShapeDtypeStruct((B, 1), jnp.float32),
        jax.ShapeDtypeStruct((B, 1), jnp.float32),
        jax.ShapeDtypeStruct((B, item), jnp.int32),
    )
    lp, rew, acts = pl.pallas_call(
        body, out_shape=out_shape,
        scratch_shapes=[pltpu.VMEM((B, NE, NE), jnp.float32),
                        pltpu.VMEM((B, NE, n4), jnp.float32)],
        interpret=interpret)(
        cce_b, cc0, cc1, cc2, cc3, costs2, G_t, init_w, Wh, bh, Wv, bv, Wq,
        bq, Wr, br, v_ptr)
    return lp[:, 0], rew[:, 0], acts


def _prep(cell_embed, original_node, num_cell, costs, init_w):
    B = num_cell.shape[0]
    item = cell_embed.shape[0] // (4 * B)
    n4 = 4 * item
    NE = cell_embed.shape[1]
    starts = jnp.concatenate(
        [jnp.zeros((1,), jnp.int32), (4 * num_cell[:-1]).astype(jnp.int32)])
    cce_b = jax.vmap(
        lambda s: lax.dynamic_slice(cell_embed, (s, 0), (n4, NE)))(starts)
    cc_b = jax.vmap(
        lambda s: lax.dynamic_slice(original_node, (s, 0), (n4, 4)))(starts)
    cc0, cc1, cc2, cc3 = (cc_b[:, :, k] for k in range(4))
    costs2 = costs.reshape(B, n4)
    base = jax.random.key(1234)
    steps = jnp.arange(B * item)
    keys = jax.vmap(lambda s: jax.random.fold_in(base, s))(steps)
    G = jax.vmap(lambda k: jax.random.gumbel(k, (n4,), jnp.float32))(keys)
    G_t = G.reshape(B, item, n4).transpose(1, 0, 2)
    return cce_b, cc0, cc1, cc2, cc3, costs2, G_t, init_w.reshape(1, -1)


def kernel(cell_embed, original_node, map, num_cell, costs, init_w, Wh, bh,
           Wv, bv, Wq, bq, Wr, br, v_ptr):
    cce_b, cc0, cc1, cc2, cc3, costs2, G_t, iw = _prep(
        cell_embed, original_node, num_cell, costs, init_w)
    lp, rew, acts = _decode_tc(
        cce_b, cc0, cc1, cc2, cc3, costs2, G_t, iw, Wh, bh.reshape(1, -1),
        Wv, bv.reshape(1, -1), Wq, bq.reshape(1, -1), Wr, br.reshape(1, -1),
        v_ptr.reshape(-1, 1))
    return lp, rew, acts
